# Initial kernel scaffold; baseline (speedup 1.0000x reference)
import jax
import jax.numpy as jnp
from jax.experimental import pallas as pl


def kernel(emb_nodes, emb_edges, pre_node_W, pre_node_b, pre_edge_W, pre_edge_b, edge_W, edge_b, node_W, node_b, lnn_g, lnn_b, lne_g, lne_b, edge_index):
    raise NotImplementedError("write your pallas kernel here")


# speedup vs baseline: 1.9971x; 1.9971x over previous
import functools

import jax
import jax.numpy as jnp
from jax import lax
from jax.experimental import pallas as pl
from jax.experimental.pallas import tpu as pltpu
from jax.experimental.pallas import tpu_sc as plsc

_N_NODES = 10000
_N_EDGES = 320000
_H = 64
_L = 5

_NC = 2
_NS = 16
_NW = _NC * _NS
_EPW = _N_EDGES // _NW
_C = 80
_NCHUNK = _EPW // _C
_N_PAD = 10240
_RPT = _N_PAD // _NS

_NBLK = 2000
_EBLK = 2000


def _elu(x):
    return jnp.where(x > 0, x, jnp.exp(jnp.minimum(x, 0.0)) - 1.0)


def _ln(x, g, b):
    m = jnp.mean(x, axis=-1, keepdims=True)
    d = x - m
    v = jnp.mean(d * d, axis=-1, keepdims=True)
    return d * lax.rsqrt(v + 1e-5) * g + b


def _dot(a, b):
    return jnp.dot(a, b, preferred_element_type=jnp.float32)


def _node_init_body(emb, pw, pb, w1, w2, v_ref, p1_ref, p2_ref):
    v = _dot(emb[...], pw[...]) + pb[...]
    v_ref[...] = v
    p1_ref[...] = _dot(v, w1[...])
    p2_ref[...] = _dot(v, w2[...])


_node_init = pl.pallas_call(
    _node_init_body,
    grid=(_N_NODES // _NBLK,),
    in_specs=[
        pl.BlockSpec((_NBLK, 128), lambda i: (i, 0)),
        pl.BlockSpec((128, _H), lambda i: (0, 0)),
        pl.BlockSpec((1, _H), lambda i: (0, 0)),
        pl.BlockSpec((_H, _H), lambda i: (0, 0)),
        pl.BlockSpec((_H, _H), lambda i: (0, 0)),
    ],
    out_specs=[pl.BlockSpec((_NBLK, _H), lambda i: (i, 0))] * 3,
    out_shape=[jax.ShapeDtypeStruct((_N_NODES, _H), jnp.float32)] * 3,
)


def _edge_init_body(emb, pw, pb, w3, eb, e_ref, ew_ref):
    e = _dot(emb[...], pw[...]) + pb[...]
    e_ref[...] = e
    ew_ref[...] = _dot(e, w3[...]) + eb[...]


_edge_init = pl.pallas_call(
    _edge_init_body,
    grid=(_N_EDGES // _EBLK,),
    in_specs=[
        pl.BlockSpec((_EBLK, 16), lambda i: (i, 0)),
        pl.BlockSpec((16, _H), lambda i: (0, 0)),
        pl.BlockSpec((1, _H), lambda i: (0, 0)),
        pl.BlockSpec((_H, _H), lambda i: (0, 0)),
        pl.BlockSpec((1, _H), lambda i: (0, 0)),
    ],
    out_specs=[pl.BlockSpec((_EBLK, _H), lambda i: (i, 0))] * 2,
    out_shape=[jax.ShapeDtypeStruct((_N_EDGES, _H), jnp.float32)] * 2,
)


def _edge_mid_body(epre, eold, g, b, w3, eb, e_ref, ew_ref):
    x = epre[...] + eold[...]
    e = _elu(_ln(x, g[...], b[...]))
    e_ref[...] = e
    ew_ref[...] = _dot(e, w3[...]) + eb[...]


_edge_mid = pl.pallas_call(
    _edge_mid_body,
    grid=(_N_EDGES // _EBLK,),
    in_specs=[
        pl.BlockSpec((_EBLK, _H), lambda i: (i, 0)),
        pl.BlockSpec((_EBLK, _H), lambda i: (i, 0)),
        pl.BlockSpec((1, _H), lambda i: (0, 0)),
        pl.BlockSpec((1, _H), lambda i: (0, 0)),
        pl.BlockSpec((_H, _H), lambda i: (0, 0)),
        pl.BlockSpec((1, _H), lambda i: (0, 0)),
    ],
    out_specs=[pl.BlockSpec((_EBLK, _H), lambda i: (i, 0))] * 2,
    out_shape=[jax.ShapeDtypeStruct((_N_EDGES, _H), jnp.float32)] * 2,
)


def _node_mid_common(aa, ab, da, db, p1, v, nw1, nw2, nb, g, b):
    deg = da[...][:, :1] + db[...][:, :1]
    aggr = aa[...] + ab[...] + deg * p1[...]
    vold = v[...]
    vnew = _dot(aggr, nw1[...]) + _dot(vold, nw2[...]) + nb[...] + vold
    return _elu(_ln(vnew, g[...], b[...]))


def _node_mid_body(aa, ab, da, db, p1, v, nw1, nw2, nb, g, b, w1n, w2n,
                   v_ref, p1_ref, p2_ref):
    vn = _node_mid_common(aa, ab, da, db, p1, v, nw1, nw2, nb, g, b)
    v_ref[...] = vn
    p1_ref[...] = _dot(vn, w1n[...])
    p2_ref[...] = _dot(vn, w2n[...])


def _node_last_body(aa, ab, da, db, p1, v, nw1, nw2, nb, g, b, v_ref):
    v_ref[...] = _node_mid_common(aa, ab, da, db, p1, v, nw1, nw2, nb, g, b)


_node_specs = [
    pl.BlockSpec((_NBLK, _H), lambda i: (i, 0)),
    pl.BlockSpec((_NBLK, _H), lambda i: (i, 0)),
    pl.BlockSpec((_NBLK, 16), lambda i: (i, 0)),
    pl.BlockSpec((_NBLK, 16), lambda i: (i, 0)),
    pl.BlockSpec((_NBLK, _H), lambda i: (i, 0)),
    pl.BlockSpec((_NBLK, _H), lambda i: (i, 0)),
    pl.BlockSpec((_H, _H), lambda i: (0, 0)),
    pl.BlockSpec((_H, _H), lambda i: (0, 0)),
    pl.BlockSpec((1, _H), lambda i: (0, 0)),
    pl.BlockSpec((1, _H), lambda i: (0, 0)),
    pl.BlockSpec((1, _H), lambda i: (0, 0)),
]

_node_mid = pl.pallas_call(
    _node_mid_body,
    grid=(_N_NODES // _NBLK,),
    in_specs=_node_specs + [
        pl.BlockSpec((_H, _H), lambda i: (0, 0)),
        pl.BlockSpec((_H, _H), lambda i: (0, 0)),
    ],
    out_specs=[pl.BlockSpec((_NBLK, _H), lambda i: (i, 0))] * 3,
    out_shape=[jax.ShapeDtypeStruct((_N_NODES, _H), jnp.float32)] * 3,
)

_node_last = pl.pallas_call(
    _node_last_body,
    grid=(_N_NODES // _NBLK,),
    in_specs=_node_specs,
    out_specs=[pl.BlockSpec((_NBLK, _H), lambda i: (i, 0))],
    out_shape=[jax.ShapeDtypeStruct((_N_NODES, _H), jnp.float32)],
)


def _build_sc(compute_epre, compute_deg):
    mesh = plsc.VectorSubcoreMesh(core_axis_name="c", subcore_axis_name="s")

    out_type = [jax.ShapeDtypeStruct((_NC * _N_PAD, _H), jnp.float32)]
    if compute_epre:
        out_type.append(jax.ShapeDtypeStruct((_N_EDGES, _H), jnp.float32))
    if compute_deg:
        out_type.append(jax.ShapeDtypeStruct((_NC * _N_PAD, 16), jnp.float32))

    scratch = [
        pltpu.VMEM((_C,), jnp.int32),
        pltpu.VMEM((_C,), jnp.int32),
        pltpu.VMEM((_C, _H), jnp.float32),
        pltpu.VMEM((_C, _H), jnp.float32),
        pltpu.VMEM_SHARED((_N_PAD, _H), jnp.float32),
        pltpu.SemaphoreType.DMA,
        pltpu.SemaphoreType.DMA,
        pltpu.SemaphoreType.DMA,
    ]
    if compute_epre:
        scratch += [
            pltpu.VMEM((_C, _H), jnp.float32),
            pltpu.VMEM((_C, _H), jnp.float32),
            pltpu.VMEM((_C, _H), jnp.float32),
        ]
    if compute_deg:
        scratch += [
            pltpu.VMEM((_C, 16), jnp.float32),
            pltpu.VMEM_SHARED((_N_PAD, 16), jnp.float32),
        ]

    def body(*refs):
        it = iter(refs)
        src_hbm = next(it)
        dst_hbm = next(it)
        ew_hbm = next(it)
        p2_hbm = next(it)
        z64_hbm = next(it)
        p1_hbm = next(it) if compute_epre else None
        z16_hbm = next(it) if compute_deg else None
        ones_hbm = next(it) if compute_deg else None
        aggr_out = next(it)
        epre_out = next(it) if compute_epre else None
        deg_out = next(it) if compute_deg else None
        srcv = next(it)
        dstv = next(it)
        ewv = next(it)
        p2sv = next(it)
        aggr_sh = next(it)
        sem1 = next(it)
        sem2 = next(it)
        sem3 = next(it)
        if compute_epre:
            p1sv = next(it)
            p2dv = next(it)
            eprev = next(it)
        if compute_deg:
            onesv = next(it)
            deg_sh = next(it)

        c = lax.axis_index("c")
        s = lax.axis_index("s")
        wid = c * _NS + s
        row0 = s * _RPT

        pltpu.sync_copy(z64_hbm, aggr_sh.at[pl.ds(row0, _RPT)])
        if compute_deg:
            pltpu.sync_copy(z16_hbm, deg_sh.at[pl.ds(row0, _RPT)])
            pltpu.sync_copy(ones_hbm, onesv)
        plsc.subcore_barrier()

        def chunk(i, carry):
            base = wid * _EPW + i * _C
            pltpu.sync_copy(src_hbm.at[pl.ds(base, _C)], srcv)
            pltpu.sync_copy(dst_hbm.at[pl.ds(base, _C)], dstv)
            pltpu.sync_copy(ew_hbm.at[pl.ds(base, _C)], ewv)
            g1 = pltpu.async_copy(p2_hbm.at[srcv], p2sv, sem1)
            if compute_epre:
                g2 = pltpu.async_copy(p1_hbm.at[srcv], p1sv, sem2)
                g3 = pltpu.async_copy(p2_hbm.at[dstv], p2dv, sem3)
            g1.wait()
            pltpu.sync_copy(ewv, aggr_sh.at[dstv], add=True)
            pltpu.sync_copy(p2sv, aggr_sh.at[dstv], add=True)
            if compute_deg:
                pltpu.sync_copy(onesv, deg_sh.at[dstv], add=True)
            if compute_epre:
                g2.wait()
                g3.wait()

                def row(r, cc):
                    for k in range(_H // 16):
                        sl = pl.ds(k * 16, 16)
                        eprev[r, sl] = ewv[r, sl] + p1sv[r, sl] + p2dv[r, sl]
                    return cc

                lax.fori_loop(0, _C, row, 0)
                pltpu.sync_copy(eprev, epre_out.at[pl.ds(base, _C)])
            return carry

        lax.fori_loop(0, _NCHUNK, chunk, 0)

        plsc.subcore_barrier()
        out0 = c * _N_PAD + row0
        pltpu.sync_copy(aggr_sh.at[pl.ds(row0, _RPT)],
                        aggr_out.at[pl.ds(out0, _RPT)])
        if compute_deg:
            pltpu.sync_copy(deg_sh.at[pl.ds(row0, _RPT)],
                            deg_out.at[pl.ds(out0, _RPT)])

    return pl.kernel(
        body, mesh=mesh, out_type=out_type, scratch_types=scratch,
        compiler_params=pltpu.CompilerParams(use_tc_tiling_on_sc=False))


_sc_first = _build_sc(compute_epre=True, compute_deg=True)
_sc_mid = _build_sc(compute_epre=True, compute_deg=False)
_sc_last = _build_sc(compute_epre=False, compute_deg=False)


def kernel(emb_nodes, emb_edges, pre_node_W, pre_node_b, pre_edge_W,
           pre_edge_b, edge_W, edge_b, node_W, node_b, lnn_g, lnn_b,
           lne_g, lne_b, edge_index):
    f32 = jnp.float32
    src = edge_index[0]
    dst = edge_index[1]

    w1 = edge_W[:, :_H, :]
    w2 = edge_W[:, _H:2 * _H, :]
    w3 = edge_W[:, 2 * _H:, :]
    nw1 = node_W[:, :_H, :]
    nw2 = node_W[:, _H:, :]

    def row(x):
        return x.reshape(1, _H)

    V, P1, P2 = _node_init(emb_nodes, pre_node_W, row(pre_node_b),
                           w1[0], w2[0])
    E, EW = _edge_init(emb_edges, pre_edge_W, row(pre_edge_b),
                       w3[0], row(edge_b[0]))

    z64 = jnp.zeros((_RPT, _H), f32)
    z16 = jnp.zeros((_RPT, 16), f32)
    ones = jnp.ones((_C, 16), f32)

    dega = degb = None
    for l in range(_L):
        last = l == _L - 1
        if l == 0:
            aggr2, epre, deg2 = _sc_first(src, dst, EW, P2, z64, P1,
                                          z16, ones)
            dega = deg2[:_N_NODES]
            degb = deg2[_N_PAD:_N_PAD + _N_NODES]
        elif not last:
            aggr2, epre = _sc_mid(src, dst, EW, P2, z64, P1)
        else:
            (aggr2,) = _sc_last(src, dst, EW, P2, z64)
            epre = None
        aggra = aggr2[:_N_NODES]
        aggrb = aggr2[_N_PAD:_N_PAD + _N_NODES]
        if not last:
            V, P1, P2 = _node_mid(aggra, aggrb, dega, degb, P1, V,
                                  nw1[l], nw2[l], row(node_b[l]),
                                  row(lnn_g[l]), row(lnn_b[l]),
                                  w1[l + 1], w2[l + 1])
            E, EW = _edge_mid(epre, E, row(lne_g[l]), row(lne_b[l]),
                              w3[l + 1], row(edge_b[l + 1]))
        else:
            (V,) = _node_last(aggra, aggrb, dega, degb, P1, V,
                              nw1[l], nw2[l], row(node_b[l]),
                              row(lnn_g[l]), row(lnn_b[l]))
    return V


# speedup vs baseline: 2.2752x; 1.1392x over previous
import functools

import jax
import jax.numpy as jnp
from jax import lax
from jax.experimental import pallas as pl
from jax.experimental.pallas import tpu as pltpu
from jax.experimental.pallas import tpu_sc as plsc

_N_NODES = 10000
_N_EDGES = 320000
_H = 64
_L = 5

_NC = 2
_NS = 16
_NW = _NC * _NS
_EPW = _N_EDGES // _NW
_C = 80
_NCHUNK = _EPW // _C
_N_PAD = 10240
_RPT = _N_PAD // _NS

_NBLK = 2000
_EBLK = 2000


def _elu(x):
    return jnp.where(x > 0, x, jnp.exp(jnp.minimum(x, 0.0)) - 1.0)


def _ln(x, g, b):
    m = jnp.mean(x, axis=-1, keepdims=True)
    d = x - m
    v = jnp.mean(d * d, axis=-1, keepdims=True)
    return d * lax.rsqrt(v + 1e-5) * g + b


def _dot(a, b):
    return jnp.dot(a, b, preferred_element_type=jnp.float32)


def _qpack(vn, w1, w2):
    p1 = _dot(vn, w1)
    p2 = _dot(vn, w2)
    return jnp.concatenate([p2, p1], axis=1), jnp.concatenate([p1, p2], axis=1)


def _node_init_body(emb, pw, pb, w1, w2, v_ref, q1_ref, q2_ref):
    v = _dot(emb[...], pw[...]) + pb[...]
    v_ref[...] = v
    q1_ref[...], q2_ref[...] = _qpack(v, w1[...], w2[...])


_node_init = pl.pallas_call(
    _node_init_body,
    grid=(_N_NODES // _NBLK,),
    in_specs=[
        pl.BlockSpec((_NBLK, 128), lambda i: (i, 0)),
        pl.BlockSpec((128, _H), lambda i: (0, 0)),
        pl.BlockSpec((1, _H), lambda i: (0, 0)),
        pl.BlockSpec((_H, _H), lambda i: (0, 0)),
        pl.BlockSpec((_H, _H), lambda i: (0, 0)),
    ],
    out_specs=[
        pl.BlockSpec((_NBLK, _H), lambda i: (i, 0)),
        pl.BlockSpec((_NBLK, 128), lambda i: (i, 0)),
        pl.BlockSpec((_NBLK, 128), lambda i: (i, 0)),
    ],
    out_shape=[
        jax.ShapeDtypeStruct((_N_NODES, _H), jnp.float32),
        jax.ShapeDtypeStruct((_N_NODES, 128), jnp.float32),
        jax.ShapeDtypeStruct((_N_NODES, 128), jnp.float32),
    ],
)


def _edge_init_body(emb, pw, pb, w3, eb, es_ref):
    e = _dot(emb[...], pw[...]) + pb[...]
    ew = _dot(e, w3[...]) + eb[...]
    es_ref[...] = jnp.concatenate([ew, e], axis=1)


_edge_init = pl.pallas_call(
    _edge_init_body,
    grid=(_N_EDGES // _EBLK,),
    in_specs=[
        pl.BlockSpec((_EBLK, 16), lambda i: (i, 0)),
        pl.BlockSpec((16, _H), lambda i: (0, 0)),
        pl.BlockSpec((1, _H), lambda i: (0, 0)),
        pl.BlockSpec((_H, _H), lambda i: (0, 0)),
        pl.BlockSpec((1, _H), lambda i: (0, 0)),
    ],
    out_specs=[pl.BlockSpec((_EBLK, 128), lambda i: (i, 0))],
    out_shape=[jax.ShapeDtypeStruct((_N_EDGES, 128), jnp.float32)],
)


def _edge_mid_body(xp, g, b, w3, eb, es_ref):
    x = xp[...][:, :_H]
    e = _elu(_ln(x, g[...], b[...]))
    es_ref[...] = jnp.concatenate([_dot(e, w3[...]) + eb[...], e], axis=1)


_edge_mid = pl.pallas_call(
    _edge_mid_body,
    grid=(_N_EDGES // _EBLK,),
    in_specs=[
        pl.BlockSpec((_EBLK, 128), lambda i: (i, 0)),
        pl.BlockSpec((1, _H), lambda i: (0, 0)),
        pl.BlockSpec((1, _H), lambda i: (0, 0)),
        pl.BlockSpec((_H, _H), lambda i: (0, 0)),
        pl.BlockSpec((1, _H), lambda i: (0, 0)),
    ],
    out_specs=[pl.BlockSpec((_EBLK, 128), lambda i: (i, 0))],
    out_shape=[jax.ShapeDtypeStruct((_N_EDGES, 128), jnp.float32)],
)


def _node_update(aa, ab, v, nw1, nw2, nb, g, b):
    aggr = aa[...][:, :_H] + ab[...][:, :_H]
    vold = v[...]
    vnew = _dot(aggr, nw1[...]) + _dot(vold, nw2[...]) + nb[...] + vold
    return _elu(_ln(vnew, g[...], b[...]))


def _node_mid_body(aa, ab, v, nw1, nw2, nb, g, b, w1n, w2n,
                   v_ref, q1_ref, q2_ref):
    vn = _node_update(aa, ab, v, nw1, nw2, nb, g, b)
    v_ref[...] = vn
    q1_ref[...], q2_ref[...] = _qpack(vn, w1n[...], w2n[...])


def _node_last_body(aa, ab, v, nw1, nw2, nb, g, b, v_ref):
    v_ref[...] = _node_update(aa, ab, v, nw1, nw2, nb, g, b)


_node_specs = [
    pl.BlockSpec((_NBLK, 128), lambda i: (i, 0)),
    pl.BlockSpec((_NBLK, 128), lambda i: (i, 0)),
    pl.BlockSpec((_NBLK, _H), lambda i: (i, 0)),
    pl.BlockSpec((_H, _H), lambda i: (0, 0)),
    pl.BlockSpec((_H, _H), lambda i: (0, 0)),
    pl.BlockSpec((1, _H), lambda i: (0, 0)),
    pl.BlockSpec((1, _H), lambda i: (0, 0)),
    pl.BlockSpec((1, _H), lambda i: (0, 0)),
]

_node_mid = pl.pallas_call(
    _node_mid_body,
    grid=(_N_NODES // _NBLK,),
    in_specs=_node_specs + [
        pl.BlockSpec((_H, _H), lambda i: (0, 0)),
        pl.BlockSpec((_H, _H), lambda i: (0, 0)),
    ],
    out_specs=[
        pl.BlockSpec((_NBLK, _H), lambda i: (i, 0)),
        pl.BlockSpec((_NBLK, 128), lambda i: (i, 0)),
        pl.BlockSpec((_NBLK, 128), lambda i: (i, 0)),
    ],
    out_shape=[
        jax.ShapeDtypeStruct((_N_NODES, _H), jnp.float32),
        jax.ShapeDtypeStruct((_N_NODES, 128), jnp.float32),
        jax.ShapeDtypeStruct((_N_NODES, 128), jnp.float32),
    ],
)

_node_last = pl.pallas_call(
    _node_last_body,
    grid=(_N_NODES // _NBLK,),
    in_specs=_node_specs,
    out_specs=[pl.BlockSpec((_NBLK, _H), lambda i: (i, 0))],
    out_shape=[jax.ShapeDtypeStruct((_N_NODES, _H), jnp.float32)],
)


def _build_sc(with_x):
    mesh = plsc.VectorSubcoreMesh(core_axis_name="c", subcore_axis_name="s")

    out_type = [jax.ShapeDtypeStruct((_NC * _N_PAD, 128), jnp.float32)]
    if with_x:
        out_type.append(jax.ShapeDtypeStruct((_N_EDGES, 128), jnp.float32))

    scratch = [
        pltpu.VMEM((_C,), jnp.int32),
        pltpu.VMEM((_C,), jnp.int32),
        pltpu.VMEM((_C, 128), jnp.float32),
        pltpu.VMEM((_C, 128), jnp.float32),
        pltpu.VMEM((_C, 128), jnp.float32),
        pltpu.VMEM_SHARED((_N_PAD, 128), jnp.float32),
        pltpu.SemaphoreType.DMA,
        pltpu.SemaphoreType.DMA,
    ]

    def body(src_hbm, dst_hbm, es_hbm, q1_hbm, q2_hbm, z_hbm, *rest):
        it = iter(rest)
        aggr_out = next(it)
        x_out = next(it) if with_x else None
        srcv = next(it)
        dstv = next(it)
        esv = next(it)
        qsv = next(it)
        qdv = next(it)
        aggr_sh = next(it)
        sem1 = next(it)
        sem2 = next(it)

        c = lax.axis_index("c")
        s = lax.axis_index("s")
        wid = c * _NS + s
        row0 = s * _RPT

        pltpu.sync_copy(z_hbm, aggr_sh.at[pl.ds(row0, _RPT)])
        plsc.subcore_barrier()

        def chunk(i, carry):
            base = wid * _EPW + i * _C
            pltpu.sync_copy(src_hbm.at[pl.ds(base, _C)], srcv)
            pltpu.sync_copy(dst_hbm.at[pl.ds(base, _C)], dstv)
            pltpu.sync_copy(es_hbm.at[pl.ds(base, _C)], esv)
            g1 = pltpu.async_copy(q1_hbm.at[srcv], qsv, sem1)
            g2 = pltpu.async_copy(q2_hbm.at[dstv], qdv, sem2)
            pltpu.sync_copy(esv, aggr_sh.at[dstv], add=True)
            g1.wait()
            pltpu.sync_copy(qsv, aggr_sh.at[dstv], add=True)
            g2.wait()
            pltpu.sync_copy(qdv, aggr_sh.at[dstv], add=True)
            if with_x:
                def row(r, cc):
                    for k in range(_H // 16):
                        sl = pl.ds(k * 16, 16)
                        sr = pl.ds(_H + k * 16, 16)
                        esv[r, sl] = (esv[r, sl] + esv[r, sr]
                                      + qsv[r, sr] + qdv[r, sr])
                    return cc

                lax.fori_loop(0, _C, row, 0)
                pltpu.sync_copy(esv, x_out.at[pl.ds(base, _C)])
            return carry

        lax.fori_loop(0, _NCHUNK, chunk, 0)

        plsc.subcore_barrier()
        out0 = c * _N_PAD + row0
        pltpu.sync_copy(aggr_sh.at[pl.ds(row0, _RPT)],
                        aggr_out.at[pl.ds(out0, _RPT)])

    return pl.kernel(body, mesh=mesh, out_type=out_type,
                     scratch_types=scratch)


_sc_mid = _build_sc(with_x=True)
_sc_last = _build_sc(with_x=False)


def kernel(emb_nodes, emb_edges, pre_node_W, pre_node_b, pre_edge_W,
           pre_edge_b, edge_W, edge_b, node_W, node_b, lnn_g, lnn_b,
           lne_g, lne_b, edge_index):
    f32 = jnp.float32
    src = edge_index[0]
    dst = edge_index[1]

    w1 = edge_W[:, :_H, :]
    w2 = edge_W[:, _H:2 * _H, :]
    w3 = edge_W[:, 2 * _H:, :]
    nw1 = node_W[:, :_H, :]
    nw2 = node_W[:, _H:, :]

    def row(x):
        return x.reshape(1, _H)

    V, Q1, Q2 = _node_init(emb_nodes, pre_node_W, row(pre_node_b),
                           w1[0], w2[0])
    (ES,) = _edge_init(emb_edges, pre_edge_W, row(pre_edge_b),
                       w3[0], row(edge_b[0]))

    z = jnp.zeros((_RPT, 128), f32)

    for l in range(_L):
        last = l == _L - 1
        if not last:
            aggr2, xp = _sc_mid(src, dst, ES, Q1, Q2, z)
        else:
            (aggr2,) = _sc_last(src, dst, ES, Q1, Q2, z)
        aggra = aggr2[:_N_NODES]
        aggrb = aggr2[_N_PAD:_N_PAD + _N_NODES]
        if not last:
            V, Q1, Q2 = _node_mid(aggra, aggrb, V, nw1[l], nw2[l],
                                  row(node_b[l]), row(lnn_g[l]),
                                  row(lnn_b[l]), w1[l + 1], w2[l + 1])
            (ES,) = _edge_mid(xp, row(lne_g[l]), row(lne_b[l]),
                              w3[l + 1], row(edge_b[l + 1]))
        else:
            (V,) = _node_last(aggra, aggrb, V, nw1[l], nw2[l],
                              row(node_b[l]), row(lnn_g[l]),
                              row(lnn_b[l]))
    return V
